# baseline (speedup 1.0000x reference)
import jax
import jax.numpy as jnp
from jax.experimental import pallas as pl

_BATCH = 4
_ROWS = 8192
_DIM = 256
_BLOCK = 2048
_NB = _ROWS // _BLOCK


def _copy_body(w_ref, out_ref):
    out_ref[...] = w_ref[...]


def kernel(tokens, W):
    del tokens
    out2d = pl.pallas_call(
        _copy_body,
        grid=(_NB, _BATCH),
        in_specs=[pl.BlockSpec((_BLOCK, _DIM), lambda i, b: (i, 0))],
        out_specs=pl.BlockSpec((_BLOCK, _DIM), lambda i, b: (b * _NB + i, 0)),
        out_shape=jax.ShapeDtypeStruct((_BATCH * _ROWS, _DIM), jnp.float32),
    )(W)
    return out2d.reshape(_BATCH, _ROWS, _DIM)
